# baseline (speedup 1.0000x reference)
import functools

import jax
import jax.numpy as jnp
from jax import lax
from jax.experimental import pallas as pl
from jax.experimental.pallas import tpu as pltpu
from jax.experimental.pallas import tpu_sc as plsc

_CH = 128
_RG = 32
_SC_CHUNK = 128


def _dot(a, b):
    return jax.lax.dot_general(a, b, (((1,), (0,)), ((), ())),
                               preferred_element_type=jnp.float32)


def _t(v):
    return jnp.transpose(v, (1, 0))


def _k1_body(gbase, x2_ref, x1_ref, iw_ref, gi_ref):
    x2b = x2_ref[0]
    x1b = x1_ref[0]
    nb = x2b.shape[1]
    s_len = x1b.shape[1]
    mm = jax.lax.dot_general(-2.0 * x2b, x1b, (((0,), (0,)), ((), ())),
                             preferred_element_type=jnp.float32)
    x2sq = jnp.sum(x2b * x2b, axis=0)[:, None]
    x1sq = jnp.sum(x1b * x1b, axis=0)[None, :]
    ws = []
    idxs = []
    for rg in range(nb // _RG):
        r0 = rg * _RG
        t1 = jnp.full((_RG, _CH), jnp.inf, jnp.float32)
        t2 = t1
        t3 = t1
        it1 = jnp.zeros((_RG, _CH), jnp.int32)
        it2 = it1
        it3 = it1
        x2sq_r = x2sq[r0:r0 + _RG]
        base_iota = lax.broadcasted_iota(jnp.int32, (_RG, _CH), 1)
        for c in range(s_len // _CH):
            lo = c * _CH
            d_c = (x2sq_r + x1sq[:, lo:lo + _CH]) + mm[r0:r0 + _RG, lo:lo + _CH]
            c1 = d_c < t1
            dsp = jnp.maximum(t1, d_c)
            dspi = jnp.where(c1, it1, c)
            t1 = jnp.minimum(t1, d_c)
            it1 = jnp.where(c1, c, it1)
            c2 = dsp < t2
            dsp2 = jnp.maximum(t2, dsp)
            dsp2i = jnp.where(c2, it2, dspi)
            t2 = jnp.minimum(t2, dsp)
            it2 = jnp.where(c2, dspi, it2)
            c3 = dsp2 < t3
            t3 = jnp.minimum(t3, dsp2)
            it3 = jnp.where(c3, dsp2i, it3)
        catv = jnp.concatenate([t1, t2, t3], axis=1)
        lane3 = jnp.concatenate([base_iota, base_iota, base_iota], axis=1)
        catc = jnp.concatenate([it1, it2, it3], axis=1)
        cati = catc * _CH + lane3
        m1 = jnp.min(catv, axis=1, keepdims=True)
        i1 = jnp.min(jnp.where(catv == m1, cati, s_len), axis=1, keepdims=True)
        cv2 = jnp.where(catv > m1, catv, jnp.inf)
        m2 = jnp.min(cv2, axis=1, keepdims=True)
        i2 = jnp.min(jnp.where(cv2 == m2, cati, s_len), axis=1, keepdims=True)
        cv3 = jnp.where(cv2 > m2, cv2, jnp.inf)
        m3 = jnp.min(cv3, axis=1, keepdims=True)
        i3 = jnp.min(jnp.where(cv3 == m3, cati, s_len), axis=1, keepdims=True)
        w1v = 1.0 / jnp.maximum(jnp.maximum(m1, 0.0), 1e-16)
        w2v = 1.0 / jnp.maximum(jnp.maximum(m2, 0.0), 1e-16)
        w3v = 1.0 / jnp.maximum(jnp.maximum(m3, 0.0), 1e-16)
        tot = w1v + w2v + w3v
        ws.append((w1v / tot, w2v / tot, w3v / tot))
        idxs.append((i1, i2, i3))
    w1 = jnp.concatenate([w[0] for w in ws], axis=0)
    w2 = jnp.concatenate([w[1] for w in ws], axis=0)
    w3 = jnp.concatenate([w[2] for w in ws], axis=0)
    i1 = jnp.concatenate([i[0] for i in idxs], axis=0)
    i2 = jnp.concatenate([i[1] for i in idxs], axis=0)
    i3 = jnp.concatenate([i[2] for i in idxs], axis=0)
    iw_ref[0] = jnp.concatenate([_t(w1), _t(w2), _t(w3)], axis=0)
    gi_ref[0] = (jnp.concatenate([_t(i1), _t(i2), _t(i3)], axis=0) + gbase)


def _k2_body(g_ref, iw_ref, p2_ref, w1a_ref, w1b_ref, b1_ref,
             y1_ref, s_ref, ss_ref):
    gb = g_ref[:, 0]
    wt = jnp.transpose(iw_ref[0], (1, 0))
    nf = (gb[0] * wt[:, 0:1] + gb[1] * wt[:, 1:2]
          + gb[2] * wt[:, 2:3])
    y1 = (_dot(nf, w1a_ref[...]) + _dot(p2_ref[0], w1b_ref[...])
          + b1_ref[...])

    @pl.when(pl.program_id(0) == 0)
    def _():
        s_ref[...] = jnp.zeros_like(s_ref)
        ss_ref[...] = jnp.zeros_like(ss_ref)

    y1_ref[0] = y1
    s_ref[...] += jnp.sum(y1, axis=0, keepdims=True)
    ss_ref[...] += jnp.sum(y1 * y1, axis=0, keepdims=True)


def _k3_body(y1_ref, a1_ref, c1_ref, w2t_ref, b2_ref, y2_ref, s_ref, ss_ref):
    z = jnp.maximum(y1_ref[0] * a1_ref[...] + c1_ref[...], 0.0)
    y2 = _dot(z, w2t_ref[...]) + b2_ref[...]

    @pl.when(pl.program_id(0) == 0)
    def _():
        s_ref[...] = jnp.zeros_like(s_ref)
        ss_ref[...] = jnp.zeros_like(ss_ref)

    y2_ref[0] = y2
    s_ref[...] += jnp.sum(y2, axis=0, keepdims=True)
    ss_ref[...] += jnp.sum(y2 * y2, axis=0, keepdims=True)


def _k4_body(y2_ref, a2_ref, c2_ref, o_ref):
    o_ref[0] = jnp.maximum(y2_ref[0] * a2_ref[...] + c2_ref[...], 0.0)


def _sc_gather(table, gidx, n_chunks):
    r = gidx.shape[0] * gidx.shape[1] * gidx.shape[2]
    c = table.shape[1]
    info = plsc.get_sparse_core_info()
    nc = info.num_cores

    mesh = plsc.VectorSubcoreMesh(core_axis_name="c", subcore_axis_name="s")

    @functools.partial(
        pl.kernel, mesh=mesh,
        out_type=jax.ShapeDtypeStruct((r, c), jnp.float32),
        scratch_types=[
            pltpu.VMEM((n_chunks, _SC_CHUNK), jnp.int32),
            pltpu.VMEM((_SC_CHUNK, c), jnp.float32),
            pltpu.VMEM((_SC_CHUNK, c), jnp.float32),
            pltpu.SemaphoreType.DMA,
            pltpu.SemaphoreType.DMA,
        ],
    )
    def k(table_hbm, idx_hbm, out_hbm, idx_v, buf0, buf1, sem0, sem1):
        wid = lax.axis_index("s") * nc + lax.axis_index("c")
        pltpu.sync_copy(idx_hbm.at[wid], idx_v)
        bufs = (buf0, buf1)
        sems = (sem0, sem1)
        h = pltpu.async_copy(table_hbm.at[idx_v.at[0]], buf0, sem0)
        for j in range(n_chunks):
            h_next = None
            if j + 1 < n_chunks:
                h_next = pltpu.async_copy(
                    table_hbm.at[idx_v.at[j + 1]],
                    bufs[(j + 1) % 2], sems[(j + 1) % 2])
            h.wait()
            base = (wid * n_chunks + j) * _SC_CHUNK
            pltpu.sync_copy(bufs[j % 2], out_hbm.at[pl.ds(base, _SC_CHUNK)])
            h = h_next

    return k(table, gidx)


def _forward(points1, points2, xyz1, xyz2, W1, b1, g1, be1, W2, b2, g2, be2):
    B, S, C = points1.shape
    N = points2.shape[1]
    H1 = W1.shape[0]
    H2 = W2.shape[0]
    Nb = min(2048, N)
    nblk = N // Nb

    x1t = jnp.transpose(xyz1, (0, 2, 1))
    x2t = jnp.transpose(xyz2, (0, 2, 1))
    w1aT = jnp.transpose(W1[:, :C])
    w1bT = jnp.transpose(W1[:, C:])
    w2T = jnp.transpose(W2)
    b1r = b1.reshape(1, H1)
    b2r = b2.reshape(1, H2)
    table = points1.reshape(B * S, C)

    nw = 32
    n_chunks = (3 * N) // (nw * _SC_CHUNK)
    Nb2 = min(2048, N)

    iws, gs = [], []
    for b in range(B):
        iw_b, gi_b = pl.pallas_call(
            functools.partial(_k1_body, b * S),
            grid=(nblk,),
            in_specs=[
                pl.BlockSpec((1, 3, Nb), lambda n, b=b: (b, 0, n)),
                pl.BlockSpec((1, 3, S), lambda n, b=b: (b, 0, 0)),
            ],
            out_specs=[
                pl.BlockSpec((1, 3, Nb), lambda n: (0, 0, n)),
                pl.BlockSpec((1, 3, Nb), lambda n: (0, 0, n)),
            ],
            out_shape=[
                jax.ShapeDtypeStruct((1, 3, N), jnp.float32),
                jax.ShapeDtypeStruct((1, 3, N), jnp.int32),
            ],
        )(x2t, x1t)
        iws.append(iw_b)
        gidx_b = gi_b.reshape(nw, n_chunks, _SC_CHUNK)
        g_b = _sc_gather(table, gidx_b, n_chunks)
        gs.append(g_b.reshape(3, 1, N, C))

    y1s, s1s, ss1s = [], [], []
    for b in range(B):
        y1_b, s1_b, ss1_b = pl.pallas_call(
            _k2_body,
            grid=(N // Nb2,),
            in_specs=[
                pl.BlockSpec((3, 1, Nb2, C), lambda n: (0, 0, n, 0)),
                pl.BlockSpec((1, 3, Nb2), lambda n: (0, 0, n)),
                pl.BlockSpec((1, Nb2, C), lambda n, b=b: (b, n, 0)),
                pl.BlockSpec((C, H1), lambda n: (0, 0)),
                pl.BlockSpec((C, H1), lambda n: (0, 0)),
                pl.BlockSpec((1, H1), lambda n: (0, 0)),
            ],
            out_specs=[
                pl.BlockSpec((1, Nb2, H1), lambda n: (0, n, 0)),
                pl.BlockSpec((1, H1), lambda n: (0, 0)),
                pl.BlockSpec((1, H1), lambda n: (0, 0)),
            ],
            out_shape=[
                jax.ShapeDtypeStruct((1, N, H1), jnp.float32),
                jax.ShapeDtypeStruct((1, H1), jnp.float32),
                jax.ShapeDtypeStruct((1, H1), jnp.float32),
            ],
        )(gs[b], iws[b], points2, w1aT, w1bT, b1r)
        y1s.append(y1_b)
        s1s.append(s1_b)
        ss1s.append(ss1_b)

    cnt = float(B * N)
    s1 = s1s[0] + s1s[1] + s1s[2] + s1s[3]
    ss1 = ss1s[0] + ss1s[1] + ss1s[2] + ss1s[3]
    mean1 = s1[0] / cnt
    var1 = ss1[0] / cnt - mean1 * mean1
    a1 = g1 / jnp.sqrt(var1 + 1e-5)
    c1 = be1 - mean1 * a1

    y2s, s2s, ss2s = [], [], []
    for b in range(B):
        y2_b, s2_b, ss2_b = pl.pallas_call(
            _k3_body,
            grid=(N // Nb2,),
            in_specs=[
                pl.BlockSpec((1, Nb2, H1), lambda n: (0, n, 0)),
                pl.BlockSpec((1, H1), lambda n: (0, 0)),
                pl.BlockSpec((1, H1), lambda n: (0, 0)),
                pl.BlockSpec((H1, H2), lambda n: (0, 0)),
                pl.BlockSpec((1, H2), lambda n: (0, 0)),
            ],
            out_specs=[
                pl.BlockSpec((1, Nb2, H2), lambda n: (0, n, 0)),
                pl.BlockSpec((1, H2), lambda n: (0, 0)),
                pl.BlockSpec((1, H2), lambda n: (0, 0)),
            ],
            out_shape=[
                jax.ShapeDtypeStruct((1, N, H2), jnp.float32),
                jax.ShapeDtypeStruct((1, H2), jnp.float32),
                jax.ShapeDtypeStruct((1, H2), jnp.float32),
            ],
        )(y1s[b], a1.reshape(1, H1), c1.reshape(1, H1), w2T, b2r)
        y2s.append(y2_b)
        s2s.append(s2_b)
        ss2s.append(ss2_b)

    s2 = s2s[0] + s2s[1] + s2s[2] + s2s[3]
    ss2 = ss2s[0] + ss2s[1] + ss2s[2] + ss2s[3]
    mean2 = s2[0] / cnt
    var2 = ss2[0] / cnt - mean2 * mean2
    a2 = g2 / jnp.sqrt(var2 + 1e-5)
    c2 = be2 - mean2 * a2

    outs = []
    for b in range(B):
        outs.append(pl.pallas_call(
            _k4_body,
            grid=(N // Nb2,),
            in_specs=[
                pl.BlockSpec((1, Nb2, H2), lambda n: (0, n, 0)),
                pl.BlockSpec((1, H2), lambda n: (0, 0)),
                pl.BlockSpec((1, H2), lambda n: (0, 0)),
            ],
            out_specs=pl.BlockSpec((1, Nb2, H2), lambda n: (0, n, 0)),
            out_shape=jax.ShapeDtypeStruct((1, N, H2), jnp.float32),
        )(y2s[b], a2.reshape(1, H2), c2.reshape(1, H2)))
    return jnp.concatenate(outs, axis=0)


def kernel(points1, points2, xyz1, xyz2, W1, b1, g1, be1, W2, b2, g2, be2):
    return _forward(points1, points2, xyz1, xyz2, W1, b1, g1, be1,
                    W2, b2, g2, be2)


# speedup vs baseline: 1.0431x; 1.0431x over previous
import functools

import jax
import jax.numpy as jnp
from jax import lax
from jax.experimental import pallas as pl
from jax.experimental.pallas import tpu as pltpu
from jax.experimental.pallas import tpu_sc as plsc

_CH = 128
_RG = 32
_SC_CHUNK = 128


def _dot(a, b):
    return jax.lax.dot_general(a, b, (((1,), (0,)), ((), ())),
                               preferred_element_type=jnp.float32)


def _t(v):
    return jnp.transpose(v, (1, 0))


def _k1_body(gbase, x2_ref, x1_ref, iw_ref, gi_ref):
    x2b = x2_ref[0]
    x1b = x1_ref[0]
    nb = x2b.shape[1]
    s_len = x1b.shape[1]
    mm = jax.lax.dot_general(-2.0 * x2b, x1b, (((0,), (0,)), ((), ())),
                             preferred_element_type=jnp.float32)
    x2sq = jnp.sum(x2b * x2b, axis=0)[:, None]
    x1sq = jnp.sum(x1b * x1b, axis=0)[None, :]
    ws = []
    idxs = []
    for rg in range(nb // _RG):
        r0 = rg * _RG
        t1 = jnp.full((_RG, _CH), jnp.inf, jnp.float32)
        t2 = t1
        t3 = t1
        it1 = jnp.zeros((_RG, _CH), jnp.int32)
        it2 = it1
        it3 = it1
        x2sq_r = x2sq[r0:r0 + _RG]
        base_iota = lax.broadcasted_iota(jnp.int32, (_RG, _CH), 1)
        for c in range(s_len // _CH):
            lo = c * _CH
            d_c = (x2sq_r + x1sq[:, lo:lo + _CH]) + mm[r0:r0 + _RG, lo:lo + _CH]
            c1 = d_c < t1
            dsp = jnp.maximum(t1, d_c)
            dspi = jnp.where(c1, it1, c)
            t1 = jnp.minimum(t1, d_c)
            it1 = jnp.where(c1, c, it1)
            c2 = dsp < t2
            dsp2 = jnp.maximum(t2, dsp)
            dsp2i = jnp.where(c2, it2, dspi)
            t2 = jnp.minimum(t2, dsp)
            it2 = jnp.where(c2, dspi, it2)
            c3 = dsp2 < t3
            t3 = jnp.minimum(t3, dsp2)
            it3 = jnp.where(c3, dsp2i, it3)
        catv = jnp.concatenate([t1, t2, t3], axis=1)
        lane3 = jnp.concatenate([base_iota, base_iota, base_iota], axis=1)
        catc = jnp.concatenate([it1, it2, it3], axis=1)
        cati = catc * _CH + lane3
        m1 = jnp.min(catv, axis=1, keepdims=True)
        i1 = jnp.min(jnp.where(catv == m1, cati, s_len), axis=1, keepdims=True)
        cv2 = jnp.where(catv > m1, catv, jnp.inf)
        m2 = jnp.min(cv2, axis=1, keepdims=True)
        i2 = jnp.min(jnp.where(cv2 == m2, cati, s_len), axis=1, keepdims=True)
        cv3 = jnp.where(cv2 > m2, cv2, jnp.inf)
        m3 = jnp.min(cv3, axis=1, keepdims=True)
        i3 = jnp.min(jnp.where(cv3 == m3, cati, s_len), axis=1, keepdims=True)
        w1v = 1.0 / jnp.maximum(jnp.maximum(m1, 0.0), 1e-16)
        w2v = 1.0 / jnp.maximum(jnp.maximum(m2, 0.0), 1e-16)
        w3v = 1.0 / jnp.maximum(jnp.maximum(m3, 0.0), 1e-16)
        tot = w1v + w2v + w3v
        ws.append((w1v / tot, w2v / tot, w3v / tot))
        idxs.append((i1, i2, i3))
    w1 = jnp.concatenate([w[0] for w in ws], axis=0)
    w2 = jnp.concatenate([w[1] for w in ws], axis=0)
    w3 = jnp.concatenate([w[2] for w in ws], axis=0)
    i1 = jnp.concatenate([i[0] for i in idxs], axis=0)
    i2 = jnp.concatenate([i[1] for i in idxs], axis=0)
    i3 = jnp.concatenate([i[2] for i in idxs], axis=0)
    iw_ref[0] = jnp.concatenate([_t(w1), _t(w2), _t(w3)], axis=0)
    gi_ref[0] = (jnp.concatenate([_t(i1), _t(i2), _t(i3)], axis=0) + gbase)


def _k2_body(g_ref, iw_ref, p2_ref, w1a_ref, w1b_ref, b1_ref,
             y1_ref, s_ref, ss_ref):
    gb = g_ref[:, 0]
    wt = jnp.transpose(iw_ref[0], (1, 0))
    nf = (gb[0] * wt[:, 0:1] + gb[1] * wt[:, 1:2]
          + gb[2] * wt[:, 2:3])
    y1 = (_dot(nf, w1a_ref[...]) + _dot(p2_ref[0], w1b_ref[...])
          + b1_ref[...])

    @pl.when(pl.program_id(0) == 0)
    def _():
        s_ref[...] = jnp.zeros_like(s_ref)
        ss_ref[...] = jnp.zeros_like(ss_ref)

    y1_ref[0] = y1
    s_ref[...] += jnp.sum(y1, axis=0, keepdims=True)
    ss_ref[...] += jnp.sum(y1 * y1, axis=0, keepdims=True)


def _k3_body(y1_ref, a1_ref, c1_ref, w2t_ref, b2_ref, y2_ref, s_ref, ss_ref):
    z = jnp.maximum(y1_ref[0] * a1_ref[...] + c1_ref[...], 0.0)
    y2 = _dot(z, w2t_ref[...]) + b2_ref[...]

    @pl.when(pl.program_id(0) == 0)
    def _():
        s_ref[...] = jnp.zeros_like(s_ref)
        ss_ref[...] = jnp.zeros_like(ss_ref)

    y2_ref[0] = y2
    s_ref[...] += jnp.sum(y2, axis=0, keepdims=True)
    ss_ref[...] += jnp.sum(y2 * y2, axis=0, keepdims=True)


def _k4_body(y2_ref, a2_ref, c2_ref, o_ref):
    o_ref[0] = jnp.maximum(y2_ref[0] * a2_ref[...] + c2_ref[...], 0.0)


def _sc_gather(table, gidx, n_chunks):
    r = gidx.shape[0] * gidx.shape[1] * gidx.shape[2]
    c = table.shape[1]
    info = plsc.get_sparse_core_info()
    nc = info.num_cores

    mesh = plsc.VectorSubcoreMesh(core_axis_name="c", subcore_axis_name="s")

    @functools.partial(
        pl.kernel, mesh=mesh,
        out_type=jax.ShapeDtypeStruct((r, c), jnp.float32),
        scratch_types=[
            pltpu.VMEM((n_chunks, _SC_CHUNK), jnp.int32),
            pltpu.VMEM((_SC_CHUNK, c), jnp.float32),
            pltpu.VMEM((_SC_CHUNK, c), jnp.float32),
            pltpu.SemaphoreType.DMA,
            pltpu.SemaphoreType.DMA,
        ],
    )
    def k(table_hbm, idx_hbm, out_hbm, idx_v, buf0, buf1, sem0, sem1):
        wid = lax.axis_index("s") * nc + lax.axis_index("c")
        pltpu.sync_copy(idx_hbm.at[wid], idx_v)
        bufs = (buf0, buf1)
        sems = (sem0, sem1)
        h = pltpu.async_copy(table_hbm.at[idx_v.at[0]], buf0, sem0)
        for j in range(n_chunks):
            h_next = None
            if j + 1 < n_chunks:
                h_next = pltpu.async_copy(
                    table_hbm.at[idx_v.at[j + 1]],
                    bufs[(j + 1) % 2], sems[(j + 1) % 2])
            h.wait()
            base = (wid * n_chunks + j) * _SC_CHUNK
            pltpu.sync_copy(bufs[j % 2], out_hbm.at[pl.ds(base, _SC_CHUNK)])
            h = h_next

    return k(table, gidx)


def _forward(points1, points2, xyz1, xyz2, W1, b1, g1, be1, W2, b2, g2, be2):
    B, S, C = points1.shape
    N = points2.shape[1]
    H1 = W1.shape[0]
    H2 = W2.shape[0]
    Nb = min(1024, N)
    nblk = N // Nb

    x1t = jnp.transpose(xyz1, (0, 2, 1))
    x2t = jnp.transpose(xyz2, (0, 2, 1))
    w1aT = jnp.transpose(W1[:, :C])
    w1bT = jnp.transpose(W1[:, C:])
    w2T = jnp.transpose(W2)
    b1r = b1.reshape(1, H1)
    b2r = b2.reshape(1, H2)
    table = points1.reshape(B * S, C)

    nw = 32
    n_chunks = (3 * N) // (nw * _SC_CHUNK)
    Nb2 = min(2048, N)

    iws, gs = [], []
    for b in range(B):
        iw_b, gi_b = pl.pallas_call(
            functools.partial(_k1_body, b * S),
            grid=(nblk,),
            in_specs=[
                pl.BlockSpec((1, 3, Nb), lambda n, b=b: (b, 0, n)),
                pl.BlockSpec((1, 3, S), lambda n, b=b: (b, 0, 0)),
            ],
            out_specs=[
                pl.BlockSpec((1, 3, Nb), lambda n: (0, 0, n)),
                pl.BlockSpec((1, 3, Nb), lambda n: (0, 0, n)),
            ],
            out_shape=[
                jax.ShapeDtypeStruct((1, 3, N), jnp.float32),
                jax.ShapeDtypeStruct((1, 3, N), jnp.int32),
            ],
        )(x2t, x1t)
        iws.append(iw_b)
        gidx_b = gi_b.reshape(nw, n_chunks, _SC_CHUNK)
        g_b = _sc_gather(table, gidx_b, n_chunks)
        gs.append(g_b.reshape(3, 1, N, C))

    y1s, s1s, ss1s = [], [], []
    for b in range(B):
        y1_b, s1_b, ss1_b = pl.pallas_call(
            _k2_body,
            grid=(N // Nb2,),
            in_specs=[
                pl.BlockSpec((3, 1, Nb2, C), lambda n: (0, 0, n, 0)),
                pl.BlockSpec((1, 3, Nb2), lambda n: (0, 0, n)),
                pl.BlockSpec((1, Nb2, C), lambda n, b=b: (b, n, 0)),
                pl.BlockSpec((C, H1), lambda n: (0, 0)),
                pl.BlockSpec((C, H1), lambda n: (0, 0)),
                pl.BlockSpec((1, H1), lambda n: (0, 0)),
            ],
            out_specs=[
                pl.BlockSpec((1, Nb2, H1), lambda n: (0, n, 0)),
                pl.BlockSpec((1, H1), lambda n: (0, 0)),
                pl.BlockSpec((1, H1), lambda n: (0, 0)),
            ],
            out_shape=[
                jax.ShapeDtypeStruct((1, N, H1), jnp.float32),
                jax.ShapeDtypeStruct((1, H1), jnp.float32),
                jax.ShapeDtypeStruct((1, H1), jnp.float32),
            ],
        )(gs[b], iws[b], points2, w1aT, w1bT, b1r)
        y1s.append(y1_b)
        s1s.append(s1_b)
        ss1s.append(ss1_b)

    cnt = float(B * N)
    s1 = s1s[0] + s1s[1] + s1s[2] + s1s[3]
    ss1 = ss1s[0] + ss1s[1] + ss1s[2] + ss1s[3]
    mean1 = s1[0] / cnt
    var1 = ss1[0] / cnt - mean1 * mean1
    a1 = g1 / jnp.sqrt(var1 + 1e-5)
    c1 = be1 - mean1 * a1

    y2s, s2s, ss2s = [], [], []
    for b in range(B):
        y2_b, s2_b, ss2_b = pl.pallas_call(
            _k3_body,
            grid=(N // Nb2,),
            in_specs=[
                pl.BlockSpec((1, Nb2, H1), lambda n: (0, n, 0)),
                pl.BlockSpec((1, H1), lambda n: (0, 0)),
                pl.BlockSpec((1, H1), lambda n: (0, 0)),
                pl.BlockSpec((H1, H2), lambda n: (0, 0)),
                pl.BlockSpec((1, H2), lambda n: (0, 0)),
            ],
            out_specs=[
                pl.BlockSpec((1, Nb2, H2), lambda n: (0, n, 0)),
                pl.BlockSpec((1, H2), lambda n: (0, 0)),
                pl.BlockSpec((1, H2), lambda n: (0, 0)),
            ],
            out_shape=[
                jax.ShapeDtypeStruct((1, N, H2), jnp.float32),
                jax.ShapeDtypeStruct((1, H2), jnp.float32),
                jax.ShapeDtypeStruct((1, H2), jnp.float32),
            ],
        )(y1s[b], a1.reshape(1, H1), c1.reshape(1, H1), w2T, b2r)
        y2s.append(y2_b)
        s2s.append(s2_b)
        ss2s.append(ss2_b)

    s2 = s2s[0] + s2s[1] + s2s[2] + s2s[3]
    ss2 = ss2s[0] + ss2s[1] + ss2s[2] + ss2s[3]
    mean2 = s2[0] / cnt
    var2 = ss2[0] / cnt - mean2 * mean2
    a2 = g2 / jnp.sqrt(var2 + 1e-5)
    c2 = be2 - mean2 * a2

    outs = []
    for b in range(B):
        outs.append(pl.pallas_call(
            _k4_body,
            grid=(N // Nb2,),
            in_specs=[
                pl.BlockSpec((1, Nb2, H2), lambda n: (0, n, 0)),
                pl.BlockSpec((1, H2), lambda n: (0, 0)),
                pl.BlockSpec((1, H2), lambda n: (0, 0)),
            ],
            out_specs=pl.BlockSpec((1, Nb2, H2), lambda n: (0, n, 0)),
            out_shape=jax.ShapeDtypeStruct((1, N, H2), jnp.float32),
        )(y2s[b], a2.reshape(1, H2), c2.reshape(1, H2)))
    return jnp.concatenate(outs, axis=0)


def kernel(points1, points2, xyz1, xyz2, W1, b1, g1, be1, W2, b2, g2, be2):
    return _forward(points1, points2, xyz1, xyz2, W1, b1, g1, be1,
                    W2, b2, g2, be2)
